# Initial kernel scaffold; baseline (speedup 1.0000x reference)
import jax
import jax.numpy as jnp
from jax.experimental import pallas as pl


def kernel(x, table):
    raise NotImplementedError("write your pallas kernel here")


# speedup vs baseline: 1.4256x; 1.4256x over previous
import jax
import jax.numpy as jnp
from jax import lax
from jax.experimental import pallas as pl
from jax.experimental.pallas import tpu as pltpu
from jax.experimental.pallas import tpu_sc as plsc

_DIM = 32
_CHUNK = 128
_NC = 2
_NS = 16
_NW = _NC * _NS


def _make_lookup(n_chunks):
  mesh = plsc.VectorSubcoreMesh(
      core_axis_name="c", subcore_axis_name="s",
      num_cores=_NC, num_subcores=_NS)

  def body(table_hbm, idx_hbm, out_hbm, idx_v, rows0, rows1, sem0, sem1):
    wid = lax.axis_index("s") * _NC + lax.axis_index("c")
    pltpu.sync_copy(idx_hbm.at[wid], idx_v)
    pltpu.async_copy(table_hbm.at[idx_v.at[0]], rows0, sem0)

    def step(i, carry):
      c = 2 * i
      pltpu.async_copy(table_hbm.at[idx_v.at[c + 1]], rows1, sem1)
      pltpu.make_async_copy(out_hbm.at[wid, c], rows0, sem0).wait()
      pltpu.sync_copy(rows0, out_hbm.at[wid, c])

      @pl.when(c + 2 < n_chunks)
      def _():
        pltpu.async_copy(table_hbm.at[idx_v.at[c + 2]], rows0, sem0)

      pltpu.make_async_copy(out_hbm.at[wid, c + 1], rows1, sem1).wait()
      pltpu.sync_copy(rows1, out_hbm.at[wid, c + 1])
      return carry

    lax.fori_loop(0, n_chunks // 2, step, 0)

  return pl.kernel(
      body,
      out_type=jax.ShapeDtypeStruct((_NW, n_chunks, _CHUNK, _DIM),
                                    jnp.float32),
      mesh=mesh,
      scratch_types=[
          pltpu.VMEM((n_chunks, _CHUNK), jnp.int32),
          pltpu.VMEM((_CHUNK, _DIM), jnp.float32),
          pltpu.VMEM((_CHUNK, _DIM), jnp.float32),
          pltpu.SemaphoreType.DMA,
          pltpu.SemaphoreType.DMA,
      ],
      compiler_params=pltpu.CompilerParams(use_tc_tiling_on_sc=False),
  )


def kernel(x, table):
  r, s = x.shape
  n_chunks = (r * s) // (_NW * _CHUNK)
  idx = x.reshape(_NW, n_chunks, _CHUNK).astype(jnp.int32)
  out = _make_lookup(n_chunks)(table, idx)
  return out.reshape(r, s, _DIM)


# speedup vs baseline: 1.4673x; 1.0293x over previous
import jax
import jax.numpy as jnp
from jax import lax
from jax.experimental import pallas as pl
from jax.experimental.pallas import tpu as pltpu
from jax.experimental.pallas import tpu_sc as plsc

_DIM = 32
_CHUNK = 128
_NBUF = 8
_NC = 2
_NS = 16
_NW = _NC * _NS


def _make_lookup(n_chunks):
  mesh = plsc.VectorSubcoreMesh(
      core_axis_name="c", subcore_axis_name="s",
      num_cores=_NC, num_subcores=_NS)

  def body(table_hbm, idx_hbm, out_hbm, idx_v, rows, gsem, ssem):
    wid = lax.axis_index("s") * _NC + lax.axis_index("c")
    pltpu.sync_copy(idx_hbm.at[wid], idx_v)
    for b in range(_NBUF):
      pltpu.async_copy(table_hbm.at[idx_v.at[b]], rows.at[b], gsem.at[b])

    def step(i, carry):
      for b in range(_NBUF):
        c = i * _NBUF + b
        pltpu.make_async_copy(out_hbm.at[wid, c], rows.at[b],
                              gsem.at[b]).wait()
        pltpu.async_copy(rows.at[b], out_hbm.at[wid, c], ssem.at[b])
      for b in range(_NBUF):
        c2 = i * _NBUF + b + _NBUF

        @pl.when(c2 < n_chunks)
        def _():
          pltpu.make_async_copy(rows.at[b], out_hbm.at[wid, c2 - _NBUF],
                                ssem.at[b]).wait()
          pltpu.async_copy(table_hbm.at[idx_v.at[c2]], rows.at[b],
                           gsem.at[b])
      return carry

    lax.fori_loop(0, n_chunks // _NBUF, step, 0)
    for b in range(_NBUF):
      c = n_chunks - _NBUF + b
      pltpu.make_async_copy(rows.at[b], out_hbm.at[wid, c], ssem.at[b]).wait()

  return pl.kernel(
      body,
      out_type=jax.ShapeDtypeStruct((_NW, n_chunks, _CHUNK, _DIM),
                                    jnp.float32),
      mesh=mesh,
      scratch_types=[
          pltpu.VMEM((n_chunks, _CHUNK), jnp.int32),
          pltpu.VMEM((_NBUF, _CHUNK, _DIM), jnp.float32),
          pltpu.SemaphoreType.DMA((_NBUF,)),
          pltpu.SemaphoreType.DMA((_NBUF,)),
      ],
      compiler_params=pltpu.CompilerParams(use_tc_tiling_on_sc=False),
  )


def kernel(x, table):
  r, s = x.shape
  n_chunks = (r * s) // (_NW * _CHUNK)
  idx = x.reshape(_NW, n_chunks, _CHUNK).astype(jnp.int32)
  out = _make_lookup(n_chunks)(table, idx)
  return out.reshape(r, s, _DIM)
